# Initial kernel scaffold; baseline (speedup 1.0000x reference)
import jax
import jax.numpy as jnp
from jax.experimental import pallas as pl


def kernel(scores_raw, bbox_deltas, im_info):
    raise NotImplementedError("write your pallas kernel here")


# speedup vs baseline: 41.3006x; 41.3006x over previous
import numpy as np
import jax
import jax.numpy as jnp
from jax.experimental import pallas as pl
from jax.experimental.pallas import tpu as pltpu

_A = 9
_STRIDE = 16
_PRE = 6000
_POST = 300
_THRESH = 0.7
_NEG = -1e30
_LANES = 128


def _anchor_planes(H, W):
    base_size = 16.0
    ratios = np.array([0.5, 1.0, 2.0])
    scales = np.array([8.0, 16.0, 32.0])
    ctr = (base_size - 1.0) / 2.0
    size = base_size * base_size
    anchors = []
    for r in ratios:
        ws = np.round(np.sqrt(size / r))
        hs = np.round(ws * r)
        for s in scales:
            w = ws * s
            h = hs * s
            anchors.append([ctr - 0.5 * (w - 1.0), ctr - 0.5 * (h - 1.0),
                            ctr + 0.5 * (w - 1.0), ctr + 0.5 * (h - 1.0)])
    base = np.asarray(anchors, dtype=np.float32)
    shift_x = np.arange(W, dtype=np.float32) * _STRIDE
    shift_y = np.arange(H, dtype=np.float32) * _STRIDE
    sx, sy = np.meshgrid(shift_x, shift_y)
    shifts = np.stack([sx.ravel(), sy.ravel(), sx.ravel(), sy.ravel()], axis=1)
    allb = (shifts[:, None, :] + base[None, :, :]).reshape(-1, 4).astype(np.float32)
    return allb[:, 0], allb[:, 1], allb[:, 2], allb[:, 3]


def _nms_kernel(dx, dy, dw, dh, sc, ax1, ay1, ax2, ay2, info, out,
                x1s, y1s, x2s, y2s, ars, scn):
    B = dx.shape[0]
    R = dx.shape[1]
    fi = (jax.lax.broadcasted_iota(jnp.int32, (R, _LANES), 0) * _LANES
          + jax.lax.broadcasted_iota(jnp.int32, (R, _LANES), 1))

    aw = ax2[...] - ax1[...] + 1.0
    ah = ay2[...] - ay1[...] + 1.0
    acx = ax1[...] + 0.5 * aw
    acy = ay1[...] + 0.5 * ah

    for b in range(B):
        pcx = dx[b] * aw + acx
        pcy = dy[b] * ah + acy
        pw = jnp.exp(dw[b]) * aw
        ph = jnp.exp(dh[b]) * ah
        hmax = info[b, 0] - 1.0
        wmax = info[b, 1] - 1.0
        x1 = jnp.clip(pcx - 0.5 * pw, 0.0, wmax)
        y1 = jnp.clip(pcy - 0.5 * ph, 0.0, hmax)
        x2 = jnp.clip(pcx + 0.5 * pw, 0.0, wmax)
        y2 = jnp.clip(pcy + 0.5 * ph, 0.0, hmax)
        x1s[b] = x1
        y1s[b] = y1
        x2s[b] = x2
        y2s[b] = y2
        ars[b] = (x2 - x1 + 1.0) * (y2 - y1 + 1.0)

        sb = sc[b]
        bits = jax.lax.bitcast_convert_type(sb, jnp.int32)
        keys = bits ^ (jax.lax.shift_right_arithmetic(bits, 31) & jnp.int32(0x7FFFFFFF))
        c0 = jnp.sum((keys >= 0).astype(jnp.int32))
        p0 = jnp.where(c0 >= _PRE, jnp.int32(0), jnp.int32(-(2**31)))

        def sel_body(t, p):
            cand = p | (jnp.int32(1) << (jnp.int32(30) - t))
            c = jnp.sum((keys >= cand).astype(jnp.int32))
            return jnp.where(c >= _PRE, cand, p)

        kth = jax.lax.fori_loop(0, 31, sel_body, p0)
        strictly = jnp.sum((keys > kth).astype(jnp.int32))
        tied = keys == kth
        t_allow = jnp.int32(_PRE) - strictly

        def idx_body(t, m):
            cand = m | (jnp.int32(1) << (jnp.int32(16) - t))
            c = jnp.sum((tied & (fi < cand)).astype(jnp.int32))
            return jnp.where(c <= t_allow, cand, m)

        mstar = jax.lax.fori_loop(0, 17, idx_body, jnp.int32(0))
        member = (keys > kth) | (tied & (fi < mstar))
        scn[b] = jnp.where(member, sb, jnp.float32(_NEG))

    fi3 = fi[None, :, :]

    def nms_body(i, _):
        v = scn[...]
        m = jnp.max(v, axis=(1, 2), keepdims=True)
        cand = jnp.where(v == m, fi3, jnp.int32(2**30))
        mi = jnp.min(cand, axis=(1, 2), keepdims=True)
        issel = fi3 == mi
        okf = (m > jnp.float32(-0.5e30)).astype(jnp.float32)
        x1v = x1s[...]
        y1v = y1s[...]
        x2v = x2s[...]
        y2v = y2s[...]
        arv = ars[...]
        zf = jnp.float32(0.0)
        bx1 = jnp.sum(jnp.where(issel, x1v, zf), axis=(1, 2), keepdims=True)
        by1 = jnp.sum(jnp.where(issel, y1v, zf), axis=(1, 2), keepdims=True)
        bx2 = jnp.sum(jnp.where(issel, x2v, zf), axis=(1, 2), keepdims=True)
        by2 = jnp.sum(jnp.where(issel, y2v, zf), axis=(1, 2), keepdims=True)
        bar = (bx2 - bx1 + 1.0) * (by2 - by1 + 1.0)
        xx1 = jnp.maximum(x1v, bx1)
        yy1 = jnp.maximum(y1v, by1)
        xx2 = jnp.minimum(x2v, bx2)
        yy2 = jnp.minimum(y2v, by2)
        iw = jnp.maximum(xx2 - xx1 + 1.0, 0.0)
        ih = jnp.maximum(yy2 - yy1 + 1.0, 0.0)
        inter = iw * ih
        iou = inter / (bar + arv - inter)
        kill = (iou > jnp.float32(_THRESH)) | issel
        scn[...] = jnp.where(kill, jnp.float32(_NEG), v)
        row = jnp.concatenate([
            (bx1 * okf)[:, 0, 0], (by1 * okf)[:, 0, 0],
            (bx2 * okf)[:, 0, 0], (by2 * okf)[:, 0, 0],
            jnp.zeros((_LANES - 4 * B,), jnp.float32)], axis=0)
        out[pl.ds(i, 1), :] = row[None, :]
        return 0

    jax.lax.fori_loop(0, _POST, nms_body, 0)


def kernel(scores_raw, bbox_deltas, im_info):
    B = scores_raw.shape[0]
    H, W = scores_raw.shape[2], scores_raw.shape[3]
    N = H * W * _A
    R = N // _LANES

    sc = jnp.transpose(scores_raw[:, _A:, :, :], (0, 2, 3, 1)).reshape(B, R, _LANES)
    d = bbox_deltas.reshape(B, _A, 4, H, W)
    dx = jnp.transpose(d[:, :, 0], (0, 2, 3, 1)).reshape(B, R, _LANES)
    dy = jnp.transpose(d[:, :, 1], (0, 2, 3, 1)).reshape(B, R, _LANES)
    dw = jnp.transpose(d[:, :, 2], (0, 2, 3, 1)).reshape(B, R, _LANES)
    dh = jnp.transpose(d[:, :, 3], (0, 2, 3, 1)).reshape(B, R, _LANES)

    a1, a2, a3, a4 = _anchor_planes(H, W)
    ax1 = jnp.asarray(a1).reshape(R, _LANES)
    ay1 = jnp.asarray(a2).reshape(R, _LANES)
    ax2 = jnp.asarray(a3).reshape(R, _LANES)
    ay2 = jnp.asarray(a4).reshape(R, _LANES)

    f32 = jnp.float32
    rows = pl.pallas_call(
        _nms_kernel,
        out_shape=jax.ShapeDtypeStruct((_POST, _LANES), f32),
        in_specs=[pl.BlockSpec(memory_space=pltpu.VMEM)] * 9
        + [pl.BlockSpec(memory_space=pltpu.SMEM)],
        out_specs=pl.BlockSpec(memory_space=pltpu.VMEM),
        scratch_shapes=[pltpu.VMEM((B, R, _LANES), f32)] * 6,
    )(dx, dy, dw, dh, sc, ax1, ay1, ax2, ay2, im_info)

    boxes = jnp.transpose(rows[:, :4 * B].reshape(_POST, 4, B), (2, 0, 1))
    bcol = jnp.broadcast_to(
        jnp.arange(B, dtype=f32)[:, None, None], (B, _POST, 1))
    return jnp.concatenate([bcol, boxes], axis=2)


# speedup vs baseline: 63.2789x; 1.5322x over previous
import numpy as np
import jax
from jax import lax
import jax.numpy as jnp
from jax.experimental import pallas as pl
from jax.experimental.pallas import tpu as pltpu
from jax.experimental.pallas import tpu_sc as plsc

_A = 9
_STRIDE = 16
_PRE = 6000
_POST = 300
_THRESH = 0.7
_NEG = -1e30
_LANES = 128
_B = 4
_N = 36864
_R = _N // _LANES
_CH = _N // 16
_CPAD = _CH + 16
_ILIST = _PRE + _CH + 16
_CR = 47
_CN = _CR * _LANES


def _anchor_planes(H, W):
    base_size = 16.0
    ratios = np.array([0.5, 1.0, 2.0])
    scales = np.array([8.0, 16.0, 32.0])
    ctr = (base_size - 1.0) / 2.0
    size = base_size * base_size
    anchors = []
    for r in ratios:
        ws = np.round(np.sqrt(size / r))
        hs = np.round(ws * r)
        for s in scales:
            w = ws * s
            h = hs * s
            anchors.append([ctr - 0.5 * (w - 1.0), ctr - 0.5 * (h - 1.0),
                            ctr + 0.5 * (w - 1.0), ctr + 0.5 * (h - 1.0)])
    base = np.asarray(anchors, dtype=np.float32)
    shift_x = np.arange(W, dtype=np.float32) * _STRIDE
    shift_y = np.arange(H, dtype=np.float32) * _STRIDE
    sx, sy = np.meshgrid(shift_x, shift_y)
    shifts = np.stack([sx.ravel(), sy.ravel(), sx.ravel(), sy.ravel()], axis=1)
    allb = (shifts[:, None, :] + base[None, :, :]).reshape(-1, 4).astype(np.float32)
    return allb[:, 0], allb[:, 1], allb[:, 2], allb[:, 3]


def _decode_kernel(dx, dy, dw, dh, sc, ax1, ay1, ax2, ay2, info,
                   planes, keys_out, thr):
    fi = (lax.broadcasted_iota(jnp.int32, (_R, _LANES), 0) * _LANES
          + lax.broadcasted_iota(jnp.int32, (_R, _LANES), 1))
    aw = ax2[...] - ax1[...] + 1.0
    ah = ay2[...] - ay1[...] + 1.0
    acx = ax1[...] + 0.5 * aw
    acy = ay1[...] + 0.5 * ah
    for b in range(_B):
        pcx = dx[b] * aw + acx
        pcy = dy[b] * ah + acy
        pw = jnp.exp(dw[b]) * aw
        ph = jnp.exp(dh[b]) * ah
        hmax = info[b, 0] - 1.0
        wmax = info[b, 1] - 1.0
        planes[0, b] = jnp.clip(pcx - 0.5 * pw, 0.0, wmax)
        planes[1, b] = jnp.clip(pcy - 0.5 * ph, 0.0, hmax)
        planes[2, b] = jnp.clip(pcx + 0.5 * pw, 0.0, wmax)
        planes[3, b] = jnp.clip(pcy + 0.5 * ph, 0.0, hmax)
        sb = sc[b]
        planes[4, b] = sb

        bits = lax.bitcast_convert_type(sb, jnp.int32)
        keys = bits ^ (lax.shift_right_arithmetic(bits, 31) & jnp.int32(0x7FFFFFFF))
        keys_out[b] = keys
        c0 = jnp.sum((keys >= 0).astype(jnp.int32))
        p0 = jnp.where(c0 >= _PRE, jnp.int32(0), jnp.int32(-(2**31)))

        def sel_body(t, p):
            cand = p | (jnp.int32(1) << (jnp.int32(30) - t))
            c = jnp.sum((keys >= cand).astype(jnp.int32))
            return jnp.where(c >= _PRE, cand, p)

        kth = lax.fori_loop(0, 31, sel_body, p0)
        strictly = jnp.sum((keys > kth).astype(jnp.int32))
        tied = keys == kth
        t_allow = jnp.int32(_PRE) - strictly

        def idx_body(t, m):
            cand = m | (jnp.int32(1) << (jnp.int32(16) - t))
            c = jnp.sum((tied & (fi < cand)).astype(jnp.int32))
            return jnp.where(c <= t_allow, cand, m)

        mstar = lax.fori_loop(0, 17, idx_body, jnp.int32(0))
        thr[2 * b, :] = jnp.full((_LANES,), kth, jnp.int32)
        thr[2 * b + 1, :] = jnp.full((_LANES,), mstar, jnp.int32)


def _compact_kernel(keys_hbm, thr_hbm, ilist_hbm,
                    kbuf, thrbuf, lbuf, cnt16, cbuf, jbuf, obuf, shloc, shcnt):
    cid = lax.axis_index("c")
    sid = lax.axis_index("s")
    for bslot in range(2):
        b = cid + 2 * bslot
        pltpu.sync_copy(keys_hbm.at[pl.ds(b * _N + sid * _CH, _CH)], kbuf)
        pltpu.sync_copy(thr_hbm.at[pl.ds(b * 2 * _LANES, 16)],
                        thrbuf.at[pl.ds(0, 16)])
        pltpu.sync_copy(thr_hbm.at[pl.ds(b * 2 * _LANES + _LANES, 16)],
                        thrbuf.at[pl.ds(16, 16)])
        kv_thr = thrbuf[pl.ds(0, 16)]
        mv_thr = thrbuf[pl.ds(16, 16)]
        chunk0 = sid * _CH

        def body(v, cur):
            kv = kbuf[pl.ds(v * 16, 16)]
            fiv = lax.iota(jnp.int32, 16) + (chunk0 + v * 16)
            mk = (kv > kv_thr) | ((kv == kv_thr) & (fiv < mv_thr))
            skey = jnp.where(mk, fiv, fiv | jnp.int32(0x40000000))
            lbuf[pl.ds(cur, 16)] = plsc.sort_key_val(skey, fiv)[1]
            c = plsc.all_reduce_population_count(mk)
            return cur + jnp.max(c)

        cnt = lax.fori_loop(0, _CH // 16, body, jnp.int32(0))
        pltpu.sync_copy(lbuf, shloc.at[pl.ds((bslot * 16 + sid) * _CPAD, _CPAD)])
        cnt16[...] = jnp.full((16,), cnt, jnp.int32)
        pltpu.sync_copy(cnt16, shcnt.at[pl.ds((bslot * 16 + sid) * 16, 16)])
    plsc.subcore_barrier()

    @pl.when(sid < 2)
    def _concat():
        b = cid + 2 * sid
        pltpu.sync_copy(shcnt.at[pl.ds(sid * 256, 256)], cbuf)

        def cbody(j, cur):
            pltpu.sync_copy(shloc.at[pl.ds((sid * 16 + j) * _CPAD, _CPAD)], jbuf)

            def copy16(i, _):
                obuf[pl.ds(cur + i * 16, 16)] = jbuf[pl.ds(i * 16, 16)]
                return 0

            lax.fori_loop(0, _CH // 16, copy16, 0)
            return cur + jnp.max(cbuf[pl.ds(j * 16, 16)])

        lax.fori_loop(0, 16, cbody, jnp.int32(0))
        pltpu.sync_copy(obuf, ilist_hbm.at[pl.ds(b * _ILIST, _ILIST)])


def _gather_kernel(planes_hbm, ilist_hbm, compact_hbm, pbuf, ibuf, obuf):
    cid = lax.axis_index("c")
    sid = lax.axis_index("s")
    wid = sid * 2 + cid

    @pl.when(wid < 5 * _B)
    def _run():
        b = wid // 5
        comp = wid - 5 * b
        pltpu.sync_copy(planes_hbm.at[pl.ds((comp * _B + b) * _N, _N)], pbuf)
        pltpu.sync_copy(ilist_hbm.at[pl.ds(b * _ILIST, _CN)], ibuf)

        def body(i, _):
            iv = ibuf[pl.ds(i * 16, 16)]
            ivc = jnp.minimum(jnp.maximum(iv, jnp.int32(0)), jnp.int32(_N - 1))
            obuf[pl.ds(i * 16, 16)] = plsc.load_gather(pbuf, [ivc])
            return 0

        lax.fori_loop(0, _CN // 16, body, 0)
        pltpu.sync_copy(obuf, compact_hbm.at[pl.ds((b * 5 + comp) * _CN, _CN)])


def _nms_kernel(cboard, out, scn, ars):
    fi = (lax.broadcasted_iota(jnp.int32, (_CR, _LANES), 0) * _LANES
          + lax.broadcasted_iota(jnp.int32, (_CR, _LANES), 1))
    fi3 = fi[None, :, :]
    c = cboard[...]
    x1v = c[:, 0]
    y1v = c[:, 1]
    x2v = c[:, 2]
    y2v = c[:, 3]
    scn[...] = jnp.where(fi3 < _PRE, c[:, 4], jnp.float32(_NEG))
    ars[...] = (x2v - x1v + 1.0) * (y2v - y1v + 1.0)

    def nms_body(i, _):
        v = scn[...]
        m = jnp.max(v, axis=(1, 2), keepdims=True)
        cand = jnp.where(v == m, fi3, jnp.int32(2**30))
        mi = jnp.min(cand, axis=(1, 2), keepdims=True)
        issel = fi3 == mi
        okf = (m > jnp.float32(-0.5e30)).astype(jnp.float32)
        arv = ars[...]
        zf = jnp.float32(0.0)
        bx1 = jnp.sum(jnp.where(issel, x1v, zf), axis=(1, 2), keepdims=True)
        by1 = jnp.sum(jnp.where(issel, y1v, zf), axis=(1, 2), keepdims=True)
        bx2 = jnp.sum(jnp.where(issel, x2v, zf), axis=(1, 2), keepdims=True)
        by2 = jnp.sum(jnp.where(issel, y2v, zf), axis=(1, 2), keepdims=True)
        bar = (bx2 - bx1 + 1.0) * (by2 - by1 + 1.0)
        xx1 = jnp.maximum(x1v, bx1)
        yy1 = jnp.maximum(y1v, by1)
        xx2 = jnp.minimum(x2v, bx2)
        yy2 = jnp.minimum(y2v, by2)
        iw = jnp.maximum(xx2 - xx1 + 1.0, 0.0)
        ih = jnp.maximum(yy2 - yy1 + 1.0, 0.0)
        inter = iw * ih
        iou = inter / (bar + arv - inter)
        kill = (iou > jnp.float32(_THRESH)) | issel
        scn[...] = jnp.where(kill, jnp.float32(_NEG), v)
        row = jnp.concatenate([
            (bx1 * okf)[:, 0, 0], (by1 * okf)[:, 0, 0],
            (bx2 * okf)[:, 0, 0], (by2 * okf)[:, 0, 0],
            jnp.zeros((_LANES - 4 * _B,), jnp.float32)], axis=0)
        out[pl.ds(i, 1), :] = row[None, :]
        return 0

    lax.fori_loop(0, _POST, nms_body, 0)


def kernel(scores_raw, bbox_deltas, im_info):
    B, H, W = scores_raw.shape[0], scores_raw.shape[2], scores_raw.shape[3]
    f32, i32 = jnp.float32, jnp.int32

    sc = jnp.transpose(scores_raw[:, _A:, :, :], (0, 2, 3, 1)).reshape(B, _R, _LANES)
    d = bbox_deltas.reshape(B, _A, 4, H, W)
    dx = jnp.transpose(d[:, :, 0], (0, 2, 3, 1)).reshape(B, _R, _LANES)
    dy = jnp.transpose(d[:, :, 1], (0, 2, 3, 1)).reshape(B, _R, _LANES)
    dw = jnp.transpose(d[:, :, 2], (0, 2, 3, 1)).reshape(B, _R, _LANES)
    dh = jnp.transpose(d[:, :, 3], (0, 2, 3, 1)).reshape(B, _R, _LANES)

    a1, a2, a3, a4 = _anchor_planes(H, W)
    ax1 = jnp.asarray(a1).reshape(_R, _LANES)
    ay1 = jnp.asarray(a2).reshape(_R, _LANES)
    ax2 = jnp.asarray(a3).reshape(_R, _LANES)
    ay2 = jnp.asarray(a4).reshape(_R, _LANES)

    planes, keys, thr = pl.pallas_call(
        _decode_kernel,
        out_shape=(jax.ShapeDtypeStruct((5, B, _R, _LANES), f32),
                   jax.ShapeDtypeStruct((B, _R, _LANES), i32),
                   jax.ShapeDtypeStruct((2 * B, _LANES), i32)),
        in_specs=[pl.BlockSpec(memory_space=pltpu.VMEM)] * 9
        + [pl.BlockSpec(memory_space=pltpu.SMEM)],
        out_specs=(pl.BlockSpec(memory_space=pltpu.VMEM),
                   pl.BlockSpec(memory_space=pltpu.VMEM),
                   pl.BlockSpec(memory_space=pltpu.VMEM)),
    )(dx, dy, dw, dh, sc, ax1, ay1, ax2, ay2, im_info)

    mesh = plsc.VectorSubcoreMesh(core_axis_name="c", subcore_axis_name="s")

    ilist = pl.kernel(
        _compact_kernel,
        mesh=mesh,
        compiler_params=pltpu.CompilerParams(needs_layout_passes=False),
        out_type=jax.ShapeDtypeStruct((B * _ILIST,), i32),
        scratch_types=[
            pltpu.VMEM((_CH,), i32),
            pltpu.VMEM((32,), i32),
            pltpu.VMEM((_CPAD,), i32),
            pltpu.VMEM((16,), i32),
            pltpu.VMEM((256,), i32),
            pltpu.VMEM((_CPAD,), i32),
            pltpu.VMEM((_ILIST,), i32),
            pltpu.VMEM_SHARED((2 * 16 * _CPAD,), i32),
            pltpu.VMEM_SHARED((2 * 16 * 16,), i32),
        ],
    )(keys.reshape(B * _N), thr.reshape(2 * B * _LANES))

    compact = pl.kernel(
        _gather_kernel,
        mesh=mesh,
        compiler_params=pltpu.CompilerParams(needs_layout_passes=False),
        out_type=jax.ShapeDtypeStruct((B * 5 * _CN,), f32),
        scratch_types=[
            pltpu.VMEM((_N,), f32),
            pltpu.VMEM((_CN,), i32),
            pltpu.VMEM((_CN,), f32),
        ],
    )(planes.reshape(5 * B * _N), ilist)

    rows = pl.pallas_call(
        _nms_kernel,
        out_shape=jax.ShapeDtypeStruct((_POST, _LANES), f32),
        in_specs=[pl.BlockSpec(memory_space=pltpu.VMEM)],
        out_specs=pl.BlockSpec(memory_space=pltpu.VMEM),
        scratch_shapes=[pltpu.VMEM((B, _CR, _LANES), f32)] * 2,
    )(compact.reshape(B, 5, _CR, _LANES))

    boxes = jnp.transpose(rows[:, :4 * B].reshape(_POST, 4, B), (2, 0, 1))
    bcol = jnp.broadcast_to(jnp.arange(B, dtype=f32)[:, None, None], (B, _POST, 1))
    return jnp.concatenate([bcol, boxes], axis=2)


# speedup vs baseline: 66.5937x; 1.0524x over previous
import numpy as np
import jax
from jax import lax
import jax.numpy as jnp
from jax.experimental import pallas as pl
from jax.experimental.pallas import tpu as pltpu
from jax.experimental.pallas import tpu_sc as plsc

_A = 9
_STRIDE = 16
_PRE = 6000
_POST = 300
_THRESH = 0.7
_NEG = -1e30
_LANES = 128
_B = 4
_N = 36864
_R = _N // _LANES
_CH = _N // 16
_CPAD = _CH + 16
_ILIST = _PRE + _CH + 16
_CR = 47
_CN = _CR * _LANES


def _anchor_planes(H, W):
    base_size = 16.0
    ratios = np.array([0.5, 1.0, 2.0])
    scales = np.array([8.0, 16.0, 32.0])
    ctr = (base_size - 1.0) / 2.0
    size = base_size * base_size
    anchors = []
    for r in ratios:
        ws = np.round(np.sqrt(size / r))
        hs = np.round(ws * r)
        for s in scales:
            w = ws * s
            h = hs * s
            anchors.append([ctr - 0.5 * (w - 1.0), ctr - 0.5 * (h - 1.0),
                            ctr + 0.5 * (w - 1.0), ctr + 0.5 * (h - 1.0)])
    base = np.asarray(anchors, dtype=np.float32)
    shift_x = np.arange(W, dtype=np.float32) * _STRIDE
    shift_y = np.arange(H, dtype=np.float32) * _STRIDE
    sx, sy = np.meshgrid(shift_x, shift_y)
    shifts = np.stack([sx.ravel(), sy.ravel(), sx.ravel(), sy.ravel()], axis=1)
    allb = (shifts[:, None, :] + base[None, :, :]).reshape(-1, 4).astype(np.float32)
    return allb[:, 0], allb[:, 1], allb[:, 2], allb[:, 3]


def _decode_kernel(dx, dy, dw, dh, sc, ax1, ay1, ax2, ay2, info,
                   planes, keys_out, thr):
    fi3 = (lax.broadcasted_iota(jnp.int32, (1, _R, _LANES), 1) * _LANES
           + lax.broadcasted_iota(jnp.int32, (1, _R, _LANES), 2))
    aw = ax2[...] - ax1[...] + 1.0
    ah = ay2[...] - ay1[...] + 1.0
    acx = ax1[...] + 0.5 * aw
    acy = ay1[...] + 0.5 * ah
    for b in range(_B):
        pcx = dx[b] * aw + acx
        pcy = dy[b] * ah + acy
        pw = jnp.exp(dw[b]) * aw
        ph = jnp.exp(dh[b]) * ah
        hmax = info[b, 0] - 1.0
        wmax = info[b, 1] - 1.0
        planes[0, b] = jnp.clip(pcx - 0.5 * pw, 0.0, wmax)
        planes[1, b] = jnp.clip(pcy - 0.5 * ph, 0.0, hmax)
        planes[2, b] = jnp.clip(pcx + 0.5 * pw, 0.0, wmax)
        planes[3, b] = jnp.clip(pcy + 0.5 * ph, 0.0, hmax)
        sb = sc[b]
        planes[4, b] = sb
        bits = lax.bitcast_convert_type(sb, jnp.int32)
        keys_out[b] = bits ^ (lax.shift_right_arithmetic(bits, 31)
                              & jnp.int32(0x7FFFFFFF))

    keys = keys_out[...]
    c0 = jnp.sum((keys >= 0).astype(jnp.int32), axis=(1, 2), keepdims=True)
    p0 = jnp.where(c0 >= _PRE, jnp.int32(0), jnp.int32(-(2**31)))

    def sel_body(t, p):
        cand = p | (jnp.int32(1) << (jnp.int32(30) - t))
        c = jnp.sum((keys >= cand).astype(jnp.int32), axis=(1, 2), keepdims=True)
        return jnp.where(c >= _PRE, cand, p)

    kth = lax.fori_loop(0, 31, sel_body, p0)
    strictly = jnp.sum((keys > kth).astype(jnp.int32), axis=(1, 2), keepdims=True)
    tied = keys == kth
    t_allow = jnp.int32(_PRE) - strictly

    def idx_body(t, m):
        cand = m | (jnp.int32(1) << (jnp.int32(16) - t))
        c = jnp.sum((tied & (fi3 < cand)).astype(jnp.int32),
                    axis=(1, 2), keepdims=True)
        return jnp.where(c <= t_allow, cand, m)

    mstar = lax.fori_loop(0, 17, idx_body, jnp.zeros((_B, 1, 1), jnp.int32))
    pair = jnp.concatenate([jnp.broadcast_to(kth, (_B, 1, _LANES)),
                            jnp.broadcast_to(mstar, (_B, 1, _LANES))], axis=1)
    thr[...] = pair.reshape(2 * _B, _LANES)


def _compact_kernel(keys_hbm, thr_hbm, ilist_hbm,
                    kbuf, thrbuf, lbuf, cnt16, cbuf, jbuf, obuf, shloc, shcnt):
    cid = lax.axis_index("c")
    sid = lax.axis_index("s")
    for bslot in range(2):
        b = cid + 2 * bslot
        pltpu.sync_copy(keys_hbm.at[pl.ds(b * _N + sid * _CH, _CH)], kbuf)
        pltpu.sync_copy(thr_hbm.at[pl.ds(b * 2 * _LANES, 16)],
                        thrbuf.at[pl.ds(0, 16)])
        pltpu.sync_copy(thr_hbm.at[pl.ds(b * 2 * _LANES + _LANES, 16)],
                        thrbuf.at[pl.ds(16, 16)])
        kv_thr = thrbuf[pl.ds(0, 16)]
        mv_thr = thrbuf[pl.ds(16, 16)]
        chunk0 = sid * _CH

        def body(v, cur):
            kv = kbuf[pl.ds(v * 16, 16)]
            fiv = lax.iota(jnp.int32, 16) + (chunk0 + v * 16)
            mk = (kv > kv_thr) | ((kv == kv_thr) & (fiv < mv_thr))
            skey = jnp.where(mk, fiv, fiv | jnp.int32(0x40000000))
            lbuf[pl.ds(cur, 16)] = plsc.sort_key_val(skey, fiv)[1]
            c = plsc.all_reduce_population_count(mk)
            return cur + jnp.max(c)

        cnt = lax.fori_loop(0, _CH // 16, body, jnp.int32(0))
        pltpu.sync_copy(lbuf, shloc.at[pl.ds((bslot * 16 + sid) * _CPAD, _CPAD)])
        cnt16[...] = jnp.full((16,), cnt, jnp.int32)
        pltpu.sync_copy(cnt16, shcnt.at[pl.ds((bslot * 16 + sid) * 16, 16)])
    plsc.subcore_barrier()

    @pl.when(sid < 2)
    def _concat():
        b = cid + 2 * sid
        pltpu.sync_copy(shcnt.at[pl.ds(sid * 256, 256)], cbuf)

        def cbody(j, cur):
            pltpu.sync_copy(shloc.at[pl.ds((sid * 16 + j) * _CPAD, _CPAD)], jbuf)

            def copy16(i, _):
                obuf[pl.ds(cur + i * 16, 16)] = jbuf[pl.ds(i * 16, 16)]
                return 0

            lax.fori_loop(0, _CH // 16, copy16, 0)
            return cur + jnp.max(cbuf[pl.ds(j * 16, 16)])

        lax.fori_loop(0, 16, cbody, jnp.int32(0))
        pltpu.sync_copy(obuf, ilist_hbm.at[pl.ds(b * _ILIST, _ILIST)])


def _gather_kernel(planes_hbm, ilist_hbm, compact_hbm, pbuf, ibuf, obuf):
    cid = lax.axis_index("c")
    sid = lax.axis_index("s")
    wid = sid * 2 + cid

    @pl.when(wid < 5 * _B)
    def _run():
        b = wid // 5
        comp = wid - 5 * b
        pltpu.sync_copy(planes_hbm.at[pl.ds((comp * _B + b) * _N, _N)], pbuf)
        pltpu.sync_copy(ilist_hbm.at[pl.ds(b * _ILIST, _CN)], ibuf)

        def body(i, _):
            iv = ibuf[pl.ds(i * 16, 16)]
            ivc = jnp.minimum(jnp.maximum(iv, jnp.int32(0)), jnp.int32(_N - 1))
            obuf[pl.ds(i * 16, 16)] = plsc.load_gather(pbuf, [ivc])
            return 0

        lax.fori_loop(0, _CN // 16, body, 0)
        pltpu.sync_copy(obuf, compact_hbm.at[pl.ds((b * 5 + comp) * _CN, _CN)])


def _nms_kernel(cboard, out):
    fi3 = (lax.broadcasted_iota(jnp.int32, (1, _CR, _LANES), 1) * _LANES
           + lax.broadcasted_iota(jnp.int32, (1, _CR, _LANES), 2))
    c = cboard[...]
    x1v = c[:, 0]
    y1v = c[:, 1]
    x2v = c[:, 2]
    y2v = c[:, 3]
    v0 = jnp.where(fi3 < _PRE, c[:, 4], jnp.float32(_NEG))
    arv = (x2v - x1v + 1.0) * (y2v - y1v + 1.0)

    def nms_body(i, v):
        m = jnp.max(v, axis=(1, 2), keepdims=True)
        cand = jnp.where(v == m, fi3, jnp.int32(2**30))
        mi = jnp.min(cand, axis=(1, 2), keepdims=True)
        issel = fi3 == mi
        okf = (m > jnp.float32(-0.5e30)).astype(jnp.float32)
        zf = jnp.float32(0.0)
        bx1 = jnp.sum(jnp.where(issel, x1v, zf), axis=(1, 2), keepdims=True)
        by1 = jnp.sum(jnp.where(issel, y1v, zf), axis=(1, 2), keepdims=True)
        bx2 = jnp.sum(jnp.where(issel, x2v, zf), axis=(1, 2), keepdims=True)
        by2 = jnp.sum(jnp.where(issel, y2v, zf), axis=(1, 2), keepdims=True)
        bar = (bx2 - bx1 + 1.0) * (by2 - by1 + 1.0)
        xx1 = jnp.maximum(x1v, bx1)
        yy1 = jnp.maximum(y1v, by1)
        xx2 = jnp.minimum(x2v, bx2)
        yy2 = jnp.minimum(y2v, by2)
        iw = jnp.maximum(xx2 - xx1 + 1.0, 0.0)
        ih = jnp.maximum(yy2 - yy1 + 1.0, 0.0)
        inter = iw * ih
        iou = inter / (bar + arv - inter)
        kill = (iou > jnp.float32(_THRESH)) | issel
        row = jnp.concatenate([
            (bx1 * okf)[:, 0, 0], (by1 * okf)[:, 0, 0],
            (bx2 * okf)[:, 0, 0], (by2 * okf)[:, 0, 0],
            jnp.zeros((_LANES - 4 * _B,), jnp.float32)], axis=0)
        out[pl.ds(i, 1), :] = row[None, :]
        return jnp.where(kill, jnp.float32(_NEG), v)

    lax.fori_loop(0, _POST, nms_body, v0)


def kernel(scores_raw, bbox_deltas, im_info):
    B, H, W = scores_raw.shape[0], scores_raw.shape[2], scores_raw.shape[3]
    f32, i32 = jnp.float32, jnp.int32

    sc = jnp.transpose(scores_raw[:, _A:, :, :], (0, 2, 3, 1)).reshape(B, _R, _LANES)
    d = bbox_deltas.reshape(B, _A, 4, H, W)
    dx = jnp.transpose(d[:, :, 0], (0, 2, 3, 1)).reshape(B, _R, _LANES)
    dy = jnp.transpose(d[:, :, 1], (0, 2, 3, 1)).reshape(B, _R, _LANES)
    dw = jnp.transpose(d[:, :, 2], (0, 2, 3, 1)).reshape(B, _R, _LANES)
    dh = jnp.transpose(d[:, :, 3], (0, 2, 3, 1)).reshape(B, _R, _LANES)

    a1, a2, a3, a4 = _anchor_planes(H, W)
    ax1 = jnp.asarray(a1).reshape(_R, _LANES)
    ay1 = jnp.asarray(a2).reshape(_R, _LANES)
    ax2 = jnp.asarray(a3).reshape(_R, _LANES)
    ay2 = jnp.asarray(a4).reshape(_R, _LANES)

    planes, keys, thr = pl.pallas_call(
        _decode_kernel,
        out_shape=(jax.ShapeDtypeStruct((5, B, _R, _LANES), f32),
                   jax.ShapeDtypeStruct((B, _R, _LANES), i32),
                   jax.ShapeDtypeStruct((2 * B, _LANES), i32)),
        in_specs=[pl.BlockSpec(memory_space=pltpu.VMEM)] * 9
        + [pl.BlockSpec(memory_space=pltpu.SMEM)],
        out_specs=(pl.BlockSpec(memory_space=pltpu.VMEM),
                   pl.BlockSpec(memory_space=pltpu.VMEM),
                   pl.BlockSpec(memory_space=pltpu.VMEM)),
    )(dx, dy, dw, dh, sc, ax1, ay1, ax2, ay2, im_info)

    mesh = plsc.VectorSubcoreMesh(core_axis_name="c", subcore_axis_name="s")

    ilist = pl.kernel(
        _compact_kernel,
        mesh=mesh,
        compiler_params=pltpu.CompilerParams(needs_layout_passes=False),
        out_type=jax.ShapeDtypeStruct((B * _ILIST,), i32),
        scratch_types=[
            pltpu.VMEM((_CH,), i32),
            pltpu.VMEM((32,), i32),
            pltpu.VMEM((_CPAD,), i32),
            pltpu.VMEM((16,), i32),
            pltpu.VMEM((256,), i32),
            pltpu.VMEM((_CPAD,), i32),
            pltpu.VMEM((_ILIST,), i32),
            pltpu.VMEM_SHARED((2 * 16 * _CPAD,), i32),
            pltpu.VMEM_SHARED((2 * 16 * 16,), i32),
        ],
    )(keys.reshape(B * _N), thr.reshape(2 * B * _LANES))

    compact = pl.kernel(
        _gather_kernel,
        mesh=mesh,
        compiler_params=pltpu.CompilerParams(needs_layout_passes=False),
        out_type=jax.ShapeDtypeStruct((B * 5 * _CN,), f32),
        scratch_types=[
            pltpu.VMEM((_N,), f32),
            pltpu.VMEM((_CN,), i32),
            pltpu.VMEM((_CN,), f32),
        ],
    )(planes.reshape(5 * B * _N), ilist)

    rows = pl.pallas_call(
        _nms_kernel,
        out_shape=jax.ShapeDtypeStruct((_POST, _LANES), f32),
        in_specs=[pl.BlockSpec(memory_space=pltpu.VMEM)],
        out_specs=pl.BlockSpec(memory_space=pltpu.VMEM),
    )(compact.reshape(B, 5, _CR, _LANES))

    boxes = jnp.transpose(rows[:, :4 * B].reshape(_POST, 4, B), (2, 0, 1))
    bcol = jnp.broadcast_to(jnp.arange(B, dtype=f32)[:, None, None], (B, _POST, 1))
    return jnp.concatenate([bcol, boxes], axis=2)


# speedup vs baseline: 83.7586x; 1.2578x over previous
import numpy as np
import jax
from jax import lax
import jax.numpy as jnp
from jax.experimental import pallas as pl
from jax.experimental.pallas import tpu as pltpu
from jax.experimental.pallas import tpu_sc as plsc

_A = 9
_STRIDE = 16
_PRE = 6000
_POST = 300
_THRESH = 0.7
_NEG = -1e30
_LANES = 128
_B = 4
_HW = 4096
_N = _A * _HW
_R = _N // _LANES
_RA = _HW // _LANES
_CH = _N // 16
_CPAD = _CH + 16
_ILIST = _PRE + _CH + 16
_CR = 47
_CN = _CR * _LANES


def _anchor_planes():
    base_size = 16.0
    ratios = np.array([0.5, 1.0, 2.0])
    scales = np.array([8.0, 16.0, 32.0])
    ctr = (base_size - 1.0) / 2.0
    size = base_size * base_size
    anchors = []
    for r in ratios:
        ws = np.round(np.sqrt(size / r))
        hs = np.round(ws * r)
        for s in scales:
            w = ws * s
            h = hs * s
            anchors.append([ctr - 0.5 * (w - 1.0), ctr - 0.5 * (h - 1.0),
                            ctr + 0.5 * (w - 1.0), ctr + 0.5 * (h - 1.0)])
    base = np.asarray(anchors, dtype=np.float32)
    shift_x = np.arange(64, dtype=np.float32) * _STRIDE
    shift_y = np.arange(64, dtype=np.float32) * _STRIDE
    sx, sy = np.meshgrid(shift_x, shift_y)
    shifts = np.stack([sx.ravel(), sy.ravel(), sx.ravel(), sy.ravel()],
                      axis=1).astype(np.float32)
    allb = (shifts[None, :, :] + base[:, None, :])
    return np.transpose(allb, (2, 0, 1)).reshape(4, _R, _LANES)


def _decode_kernel(scr, dlt, anc, info, planes, keys_out, thr):
    p3 = (lax.broadcasted_iota(jnp.int32, (1, _R, _LANES), 1) * _LANES
          + lax.broadcasted_iota(jnp.int32, (1, _R, _LANES), 2))
    n3 = (p3 & jnp.int32(_HW - 1)) * _A + lax.shift_right_logical(p3, 12)

    ablk = []
    for a in range(_A):
        ax1 = anc[0, pl.ds(_RA * a, _RA)]
        ay1 = anc[1, pl.ds(_RA * a, _RA)]
        ax2 = anc[2, pl.ds(_RA * a, _RA)]
        ay2 = anc[3, pl.ds(_RA * a, _RA)]
        aw = ax2 - ax1 + 1.0
        ah = ay2 - ay1 + 1.0
        ablk.append((aw, ah, ax1 + 0.5 * aw, ay1 + 0.5 * ah))

    for b in range(_B):
        hmax = info[b, 0] - 1.0
        wmax = info[b, 1] - 1.0
        for a in range(_A):
            aw, ah, acx, acy = ablk[a]
            pcx = dlt[b, 4 * a + 0] * aw + acx
            pcy = dlt[b, 4 * a + 1] * ah + acy
            pw = jnp.exp(dlt[b, 4 * a + 2]) * aw
            ph = jnp.exp(dlt[b, 4 * a + 3]) * ah
            rs = pl.ds(_RA * a, _RA)
            planes[0, b, rs] = jnp.clip(pcx - 0.5 * pw, 0.0, wmax)
            planes[1, b, rs] = jnp.clip(pcy - 0.5 * ph, 0.0, hmax)
            planes[2, b, rs] = jnp.clip(pcx + 0.5 * pw, 0.0, wmax)
            planes[3, b, rs] = jnp.clip(pcy + 0.5 * ph, 0.0, hmax)
            sb = scr[b, _A + a]
            planes[4, b, rs] = sb
            bits = lax.bitcast_convert_type(sb, jnp.int32)
            keys_out[b, rs] = bits ^ (lax.shift_right_arithmetic(bits, 31)
                                      & jnp.int32(0x7FFFFFFF))

    keys = keys_out[...]
    c0 = jnp.sum((keys >= 0).astype(jnp.int32), axis=(1, 2), keepdims=True)
    p0 = jnp.where(c0 >= _PRE, jnp.int32(0), jnp.int32(-(2**31)))

    def sel_body(t, p):
        cand = p | (jnp.int32(1) << (jnp.int32(30) - t))
        c = jnp.sum((keys >= cand).astype(jnp.int32), axis=(1, 2), keepdims=True)
        return jnp.where(c >= _PRE, cand, p)

    kth = lax.fori_loop(0, 31, sel_body, p0)
    strictly = jnp.sum((keys > kth).astype(jnp.int32), axis=(1, 2), keepdims=True)
    tied = keys == kth
    t_allow = jnp.int32(_PRE) - strictly

    def idx_body(t, m):
        cand = m | (jnp.int32(1) << (jnp.int32(16) - t))
        c = jnp.sum((tied & (n3 < cand)).astype(jnp.int32),
                    axis=(1, 2), keepdims=True)
        return jnp.where(c <= t_allow, cand, m)

    mstar = lax.fori_loop(0, 17, idx_body, jnp.zeros((_B, 1, 1), jnp.int32))
    pair = jnp.concatenate([jnp.broadcast_to(kth, (_B, 1, _LANES)),
                            jnp.broadcast_to(mstar, (_B, 1, _LANES))], axis=1)
    thr[...] = pair.reshape(2 * _B, _LANES)


def _compact_kernel(keys_hbm, thr_hbm, ilist_hbm,
                    kbuf, thrbuf, lbuf, cnt16, cbuf, jbuf, obuf, shloc, shcnt):
    cid = lax.axis_index("c")
    sid = lax.axis_index("s")
    for bslot in range(2):
        b = cid + 2 * bslot
        pltpu.sync_copy(keys_hbm.at[pl.ds(b * _N + sid * _CH, _CH)], kbuf)
        pltpu.sync_copy(thr_hbm.at[pl.ds(b * 2 * _LANES, 16)],
                        thrbuf.at[pl.ds(0, 16)])
        pltpu.sync_copy(thr_hbm.at[pl.ds(b * 2 * _LANES + _LANES, 16)],
                        thrbuf.at[pl.ds(16, 16)])
        kv_thr = thrbuf[pl.ds(0, 16)]
        mv_thr = thrbuf[pl.ds(16, 16)]
        chunk0 = sid * _CH

        def body(v, cur):
            kv = kbuf[pl.ds(v * 16, 16)]
            fiv = lax.iota(jnp.int32, 16) + (chunk0 + v * 16)
            nv = ((fiv & jnp.int32(_HW - 1)) * _A
                  + lax.shift_right_logical(fiv, 12))
            mk = (kv > kv_thr) | ((kv == kv_thr) & (nv < mv_thr))
            skey = jnp.where(mk, fiv, fiv | jnp.int32(0x40000000))
            lbuf[pl.ds(cur, 16)] = plsc.sort_key_val(skey, fiv)[1]
            c = plsc.all_reduce_population_count(mk)
            return cur + jnp.max(c)

        cnt = lax.fori_loop(0, _CH // 16, body, jnp.int32(0))
        pltpu.sync_copy(lbuf, shloc.at[pl.ds((bslot * 16 + sid) * _CPAD, _CPAD)])
        cnt16[...] = jnp.full((16,), cnt, jnp.int32)
        pltpu.sync_copy(cnt16, shcnt.at[pl.ds((bslot * 16 + sid) * 16, 16)])
    plsc.subcore_barrier()

    @pl.when(sid < 2)
    def _concat():
        b = cid + 2 * sid
        pltpu.sync_copy(shcnt.at[pl.ds(sid * 256, 256)], cbuf)

        def cbody(j, cur):
            pltpu.sync_copy(shloc.at[pl.ds((sid * 16 + j) * _CPAD, _CPAD)], jbuf)

            def copy16(i, _):
                obuf[pl.ds(cur + i * 16, 16)] = jbuf[pl.ds(i * 16, 16)]
                return 0

            lax.fori_loop(0, _CH // 16, copy16, 0)
            return cur + jnp.max(cbuf[pl.ds(j * 16, 16)])

        lax.fori_loop(0, 16, cbody, jnp.int32(0))
        pltpu.sync_copy(obuf, ilist_hbm.at[pl.ds(b * _ILIST, _ILIST)])


def _gather_kernel(planes_hbm, ilist_hbm, compact_hbm, pbuf, ibuf, obuf):
    cid = lax.axis_index("c")
    sid = lax.axis_index("s")
    wid = sid * 2 + cid

    @pl.when(wid < 5 * _B)
    def _run():
        b = wid // 5
        comp = wid - 5 * b
        pltpu.sync_copy(planes_hbm.at[pl.ds((comp * _B + b) * _N, _N)], pbuf)
        pltpu.sync_copy(ilist_hbm.at[pl.ds(b * _ILIST, _CN)], ibuf)

        def body(i, _):
            iv = ibuf[pl.ds(i * 16, 16)]
            ivc = jnp.minimum(jnp.maximum(iv, jnp.int32(0)), jnp.int32(_N - 1))
            obuf[pl.ds(i * 16, 16)] = plsc.load_gather(pbuf, [ivc])
            return 0

        lax.fori_loop(0, _CN // 16, body, 0)
        pltpu.sync_copy(obuf, compact_hbm.at[pl.ds((b * 5 + comp) * _CN, _CN)])


def _nms_kernel(cboard, pcomp, out):
    fi3 = (lax.broadcasted_iota(jnp.int32, (1, _CR, _LANES), 1) * _LANES
           + lax.broadcasted_iota(jnp.int32, (1, _CR, _LANES), 2))
    c = cboard[...]
    x1v = c[:, 0]
    y1v = c[:, 1]
    x2v = c[:, 2]
    y2v = c[:, 3]
    pv = pcomp[...]
    nc = jnp.where(fi3 < _PRE,
                   (pv & jnp.int32(_HW - 1)) * _A + lax.shift_right_logical(pv, 12),
                   jnp.int32(2**29))
    v0 = jnp.where(fi3 < _PRE, c[:, 4], jnp.float32(_NEG))
    arv = (x2v - x1v + 1.0) * (y2v - y1v + 1.0)

    def nms_body(i, v):
        m = jnp.max(v, axis=(1, 2), keepdims=True)
        cand = jnp.where(v == m, nc, jnp.int32(2**30))
        mi = jnp.min(cand, axis=(1, 2), keepdims=True)
        issel = nc == mi
        okf = (m > jnp.float32(-0.5e30)).astype(jnp.float32)
        zf = jnp.float32(0.0)
        bx1 = jnp.sum(jnp.where(issel, x1v, zf), axis=(1, 2), keepdims=True)
        by1 = jnp.sum(jnp.where(issel, y1v, zf), axis=(1, 2), keepdims=True)
        bx2 = jnp.sum(jnp.where(issel, x2v, zf), axis=(1, 2), keepdims=True)
        by2 = jnp.sum(jnp.where(issel, y2v, zf), axis=(1, 2), keepdims=True)
        bar = (bx2 - bx1 + 1.0) * (by2 - by1 + 1.0)
        xx1 = jnp.maximum(x1v, bx1)
        yy1 = jnp.maximum(y1v, by1)
        xx2 = jnp.minimum(x2v, bx2)
        yy2 = jnp.minimum(y2v, by2)
        iw = jnp.maximum(xx2 - xx1 + 1.0, 0.0)
        ih = jnp.maximum(yy2 - yy1 + 1.0, 0.0)
        inter = iw * ih
        iou = inter / (bar + arv - inter)
        kill = (iou > jnp.float32(_THRESH)) | issel
        row = jnp.concatenate([
            (bx1 * okf)[:, 0, 0], (by1 * okf)[:, 0, 0],
            (bx2 * okf)[:, 0, 0], (by2 * okf)[:, 0, 0],
            jnp.zeros((_LANES - 4 * _B,), jnp.float32)], axis=0)
        out[pl.ds(i, 1), :] = row[None, :]
        return jnp.where(kill, jnp.float32(_NEG), v)

    lax.fori_loop(0, _POST, nms_body, v0)


def kernel(scores_raw, bbox_deltas, im_info):
    B = scores_raw.shape[0]
    f32, i32 = jnp.float32, jnp.int32

    scr = scores_raw.reshape(B, 2 * _A, _RA, _LANES)
    dlt = bbox_deltas.reshape(B, 4 * _A, _RA, _LANES)
    anc = jnp.asarray(_anchor_planes())

    planes, keys, thr = pl.pallas_call(
        _decode_kernel,
        out_shape=(jax.ShapeDtypeStruct((5, B, _R, _LANES), f32),
                   jax.ShapeDtypeStruct((B, _R, _LANES), i32),
                   jax.ShapeDtypeStruct((2 * B, _LANES), i32)),
        in_specs=[pl.BlockSpec(memory_space=pltpu.VMEM)] * 3
        + [pl.BlockSpec(memory_space=pltpu.SMEM)],
        out_specs=(pl.BlockSpec(memory_space=pltpu.VMEM),
                   pl.BlockSpec(memory_space=pltpu.VMEM),
                   pl.BlockSpec(memory_space=pltpu.VMEM)),
    )(scr, dlt, anc, im_info)

    mesh = plsc.VectorSubcoreMesh(core_axis_name="c", subcore_axis_name="s")

    ilist = pl.kernel(
        _compact_kernel,
        mesh=mesh,
        compiler_params=pltpu.CompilerParams(needs_layout_passes=False),
        out_type=jax.ShapeDtypeStruct((B * _ILIST,), i32),
        scratch_types=[
            pltpu.VMEM((_CH,), i32),
            pltpu.VMEM((32,), i32),
            pltpu.VMEM((_CPAD,), i32),
            pltpu.VMEM((16,), i32),
            pltpu.VMEM((256,), i32),
            pltpu.VMEM((_CPAD,), i32),
            pltpu.VMEM((_ILIST,), i32),
            pltpu.VMEM_SHARED((2 * 16 * _CPAD,), i32),
            pltpu.VMEM_SHARED((2 * 16 * 16,), i32),
        ],
    )(keys.reshape(B * _N), thr.reshape(2 * B * _LANES))

    compact = pl.kernel(
        _gather_kernel,
        mesh=mesh,
        compiler_params=pltpu.CompilerParams(needs_layout_passes=False),
        out_type=jax.ShapeDtypeStruct((B * 5 * _CN,), f32),
        scratch_types=[
            pltpu.VMEM((_N,), f32),
            pltpu.VMEM((_CN,), i32),
            pltpu.VMEM((_CN,), f32),
        ],
    )(planes.reshape(5 * B * _N), ilist)

    pc = ilist.reshape(B, _ILIST)[:, :_CN].reshape(B, _CR, _LANES)

    rows = pl.pallas_call(
        _nms_kernel,
        out_shape=jax.ShapeDtypeStruct((_POST, _LANES), f32),
        in_specs=[pl.BlockSpec(memory_space=pltpu.VMEM)] * 2,
        out_specs=pl.BlockSpec(memory_space=pltpu.VMEM),
    )(compact.reshape(B, 5, _CR, _LANES), pc)

    boxes = jnp.transpose(rows[:, :4 * B].reshape(_POST, 4, B), (2, 0, 1))
    bcol = jnp.broadcast_to(jnp.arange(B, dtype=f32)[:, None, None], (B, _POST, 1))
    return jnp.concatenate([bcol, boxes], axis=2)


# speedup vs baseline: 84.2480x; 1.0058x over previous
import numpy as np
import jax
from jax import lax
import jax.numpy as jnp
from jax.experimental import pallas as pl
from jax.experimental.pallas import tpu as pltpu
from jax.experimental.pallas import tpu_sc as plsc

_A = 9
_STRIDE = 16
_PRE = 6000
_POST = 300
_THRESH = 0.7
_NEG = -1e30
_LANES = 128
_B = 4
_HW = 4096
_N = _A * _HW
_R = _N // _LANES
_RA = _HW // _LANES
_CH = _N // 16
_CPAD = _CH + 16
_ILIST = _PRE + _CH + 16
_CR = 47
_CN = _CR * _LANES


def _anchor_planes():
    base_size = 16.0
    ratios = np.array([0.5, 1.0, 2.0])
    scales = np.array([8.0, 16.0, 32.0])
    ctr = (base_size - 1.0) / 2.0
    size = base_size * base_size
    anchors = []
    for r in ratios:
        ws = np.round(np.sqrt(size / r))
        hs = np.round(ws * r)
        for s in scales:
            w = ws * s
            h = hs * s
            anchors.append([ctr - 0.5 * (w - 1.0), ctr - 0.5 * (h - 1.0),
                            ctr + 0.5 * (w - 1.0), ctr + 0.5 * (h - 1.0)])
    base = np.asarray(anchors, dtype=np.float32)
    shift_x = np.arange(64, dtype=np.float32) * _STRIDE
    shift_y = np.arange(64, dtype=np.float32) * _STRIDE
    sx, sy = np.meshgrid(shift_x, shift_y)
    shifts = np.stack([sx.ravel(), sy.ravel(), sx.ravel(), sy.ravel()],
                      axis=1).astype(np.float32)
    allb = (shifts[None, :, :] + base[:, None, :])
    return np.transpose(allb, (2, 0, 1)).reshape(4, _R, _LANES)


def _decode_kernel(scr, dlt, anc, info, planes, keys_out, thr):
    p3 = (lax.broadcasted_iota(jnp.int32, (1, _R, _LANES), 1) * _LANES
          + lax.broadcasted_iota(jnp.int32, (1, _R, _LANES), 2))
    n3 = (p3 & jnp.int32(_HW - 1)) * _A + lax.shift_right_logical(p3, 12)

    ablk = []
    for a in range(_A):
        ax1 = anc[0, pl.ds(_RA * a, _RA)]
        ay1 = anc[1, pl.ds(_RA * a, _RA)]
        ax2 = anc[2, pl.ds(_RA * a, _RA)]
        ay2 = anc[3, pl.ds(_RA * a, _RA)]
        aw = ax2 - ax1 + 1.0
        ah = ay2 - ay1 + 1.0
        ablk.append((aw, ah, ax1 + 0.5 * aw, ay1 + 0.5 * ah))

    for b in range(_B):
        hmax = info[b, 0] - 1.0
        wmax = info[b, 1] - 1.0
        for a in range(_A):
            aw, ah, acx, acy = ablk[a]
            pcx = dlt[b, 4 * a + 0] * aw + acx
            pcy = dlt[b, 4 * a + 1] * ah + acy
            pw = jnp.exp(dlt[b, 4 * a + 2]) * aw
            ph = jnp.exp(dlt[b, 4 * a + 3]) * ah
            rs = pl.ds(_RA * a, _RA)
            planes[0, b, rs] = jnp.clip(pcx - 0.5 * pw, 0.0, wmax)
            planes[1, b, rs] = jnp.clip(pcy - 0.5 * ph, 0.0, hmax)
            planes[2, b, rs] = jnp.clip(pcx + 0.5 * pw, 0.0, wmax)
            planes[3, b, rs] = jnp.clip(pcy + 0.5 * ph, 0.0, hmax)
            sb = scr[b, _A + a]
            planes[4, b, rs] = sb
            bits = lax.bitcast_convert_type(sb, jnp.int32)
            keys_out[b, rs] = bits ^ (lax.shift_right_arithmetic(bits, 31)
                                      & jnp.int32(0x7FFFFFFF))

    keys = keys_out[...]
    c0 = jnp.sum((keys >= 0).astype(jnp.int32), axis=(1, 2), keepdims=True)
    p0 = jnp.where(c0 >= _PRE, jnp.int32(0), jnp.int32(-(2**31)))

    def sel_body(t, p):
        cand = p | (jnp.int32(1) << (jnp.int32(30) - t))
        c = jnp.sum((keys >= cand).astype(jnp.int32), axis=(1, 2), keepdims=True)
        return jnp.where(c >= _PRE, cand, p)

    kth = lax.fori_loop(0, 31, sel_body, p0)
    strictly = jnp.sum((keys > kth).astype(jnp.int32), axis=(1, 2), keepdims=True)
    tied = keys == kth
    t_allow = jnp.int32(_PRE) - strictly

    def idx_body(t, m):
        cand = m | (jnp.int32(1) << (jnp.int32(16) - t))
        c = jnp.sum((tied & (n3 < cand)).astype(jnp.int32),
                    axis=(1, 2), keepdims=True)
        return jnp.where(c <= t_allow, cand, m)

    mstar = lax.fori_loop(0, 17, idx_body, jnp.zeros((_B, 1, 1), jnp.int32))
    pair = jnp.concatenate([jnp.broadcast_to(kth, (_B, 1, _LANES)),
                            jnp.broadcast_to(mstar, (_B, 1, _LANES))], axis=1)
    thr[...] = pair.reshape(2 * _B, _LANES)


def _compact_kernel(keys_hbm, thr_hbm, ilist_hbm,
                    kbuf, thrbuf, lbuf, cnt16, cbuf, jbuf, obuf, shloc, shcnt):
    cid = lax.axis_index("c")
    sid = lax.axis_index("s")
    for bslot in range(2):
        b = cid + 2 * bslot
        pltpu.sync_copy(keys_hbm.at[pl.ds(b * _N + sid * _CH, _CH)], kbuf)
        pltpu.sync_copy(thr_hbm.at[pl.ds(b * 2 * _LANES, 16)],
                        thrbuf.at[pl.ds(0, 16)])
        pltpu.sync_copy(thr_hbm.at[pl.ds(b * 2 * _LANES + _LANES, 16)],
                        thrbuf.at[pl.ds(16, 16)])
        kv_thr = thrbuf[pl.ds(0, 16)]
        mv_thr = thrbuf[pl.ds(16, 16)]
        chunk0 = sid * _CH

        def body(v, cur):
            kv = kbuf[pl.ds(v * 16, 16)]
            fiv = lax.iota(jnp.int32, 16) + (chunk0 + v * 16)
            nv = ((fiv & jnp.int32(_HW - 1)) * _A
                  + lax.shift_right_logical(fiv, 12))
            mk = (kv > kv_thr) | ((kv == kv_thr) & (nv < mv_thr))
            skey = jnp.where(mk, fiv, fiv | jnp.int32(0x40000000))
            lbuf[pl.ds(cur, 16)] = plsc.sort_key_val(skey, fiv)[1]
            c = plsc.all_reduce_population_count(mk)
            return cur + jnp.max(c)

        cnt = lax.fori_loop(0, _CH // 16, body, jnp.int32(0))
        pltpu.sync_copy(lbuf, shloc.at[pl.ds((bslot * 16 + sid) * _CPAD, _CPAD)])
        cnt16[...] = jnp.full((16,), cnt, jnp.int32)
        pltpu.sync_copy(cnt16, shcnt.at[pl.ds((bslot * 16 + sid) * 16, 16)])
    plsc.subcore_barrier()

    @pl.when(sid < 2)
    def _concat():
        b = cid + 2 * sid
        pltpu.sync_copy(shcnt.at[pl.ds(sid * 256, 256)], cbuf)

        def cbody(j, cur):
            pltpu.sync_copy(shloc.at[pl.ds((sid * 16 + j) * _CPAD, _CPAD)], jbuf)

            def copy16(i, _):
                for u in range(4):
                    off = i * 64 + u * 16
                    obuf[pl.ds(cur + off, 16)] = jbuf[pl.ds(off, 16)]
                return 0

            lax.fori_loop(0, _CH // 64, copy16, 0)
            return cur + jnp.max(cbuf[pl.ds(j * 16, 16)])

        lax.fori_loop(0, 16, cbody, jnp.int32(0))
        pltpu.sync_copy(obuf, ilist_hbm.at[pl.ds(b * _ILIST, _ILIST)])


def _gather_kernel(planes_hbm, ilist_hbm, compact_hbm, pbuf, ibuf, obuf):
    cid = lax.axis_index("c")
    sid = lax.axis_index("s")
    wid = sid * 2 + cid

    @pl.when(wid < 5 * _B)
    def _run():
        b = wid // 5
        comp = wid - 5 * b
        pltpu.sync_copy(planes_hbm.at[pl.ds((comp * _B + b) * _N, _N)], pbuf)
        pltpu.sync_copy(ilist_hbm.at[pl.ds(b * _ILIST, _CN)], ibuf)

        def body(i, _):
            for u in range(4):
                off = i * 64 + u * 16
                iv = ibuf[pl.ds(off, 16)]
                ivc = jnp.minimum(jnp.maximum(iv, jnp.int32(0)),
                                  jnp.int32(_N - 1))
                obuf[pl.ds(off, 16)] = plsc.load_gather(pbuf, [ivc])
            return 0

        lax.fori_loop(0, _CN // 64, body, 0)
        pltpu.sync_copy(obuf, compact_hbm.at[pl.ds((b * 5 + comp) * _CN, _CN)])


def _nms_kernel(cboard, pcomp, out):
    fi3 = (lax.broadcasted_iota(jnp.int32, (1, _CR, _LANES), 1) * _LANES
           + lax.broadcasted_iota(jnp.int32, (1, _CR, _LANES), 2))
    c = cboard[...]
    x1v = c[:, 0]
    y1v = c[:, 1]
    x2v = c[:, 2]
    y2v = c[:, 3]
    pv = pcomp[...]
    nc = jnp.where(fi3 < _PRE,
                   (pv & jnp.int32(_HW - 1)) * _A + lax.shift_right_logical(pv, 12),
                   jnp.int32(2**29))
    v0 = jnp.where(fi3 < _PRE, c[:, 4], jnp.float32(_NEG))
    arv = (x2v - x1v + 1.0) * (y2v - y1v + 1.0)

    def nms_body(i, v):
        m = jnp.max(v, axis=(1, 2), keepdims=True)
        cand = jnp.where(v == m, nc, jnp.int32(2**30))
        mi = jnp.min(cand, axis=(1, 2), keepdims=True)
        issel = nc == mi
        okf = (m > jnp.float32(-0.5e30)).astype(jnp.float32)
        zf = jnp.float32(0.0)
        bx1 = jnp.sum(jnp.where(issel, x1v, zf), axis=(1, 2), keepdims=True)
        by1 = jnp.sum(jnp.where(issel, y1v, zf), axis=(1, 2), keepdims=True)
        bx2 = jnp.sum(jnp.where(issel, x2v, zf), axis=(1, 2), keepdims=True)
        by2 = jnp.sum(jnp.where(issel, y2v, zf), axis=(1, 2), keepdims=True)
        bar = (bx2 - bx1 + 1.0) * (by2 - by1 + 1.0)
        xx1 = jnp.maximum(x1v, bx1)
        yy1 = jnp.maximum(y1v, by1)
        xx2 = jnp.minimum(x2v, bx2)
        yy2 = jnp.minimum(y2v, by2)
        iw = jnp.maximum(xx2 - xx1 + 1.0, 0.0)
        ih = jnp.maximum(yy2 - yy1 + 1.0, 0.0)
        inter = iw * ih
        iou = inter / (bar + arv - inter)
        kill = (iou > jnp.float32(_THRESH)) | issel
        row = jnp.concatenate([
            (bx1 * okf)[:, 0, 0], (by1 * okf)[:, 0, 0],
            (bx2 * okf)[:, 0, 0], (by2 * okf)[:, 0, 0],
            jnp.zeros((_LANES - 4 * _B,), jnp.float32)], axis=0)
        out[pl.ds(i, 1), :] = row[None, :]
        return jnp.where(kill, jnp.float32(_NEG), v)

    lax.fori_loop(0, _POST, nms_body, v0)


def kernel(scores_raw, bbox_deltas, im_info):
    B = scores_raw.shape[0]
    f32, i32 = jnp.float32, jnp.int32

    scr = scores_raw.reshape(B, 2 * _A, _RA, _LANES)
    dlt = bbox_deltas.reshape(B, 4 * _A, _RA, _LANES)
    anc = jnp.asarray(_anchor_planes())

    planes, keys, thr = pl.pallas_call(
        _decode_kernel,
        out_shape=(jax.ShapeDtypeStruct((5, B, _R, _LANES), f32),
                   jax.ShapeDtypeStruct((B, _R, _LANES), i32),
                   jax.ShapeDtypeStruct((2 * B, _LANES), i32)),
        in_specs=[pl.BlockSpec(memory_space=pltpu.VMEM)] * 3
        + [pl.BlockSpec(memory_space=pltpu.SMEM)],
        out_specs=(pl.BlockSpec(memory_space=pltpu.VMEM),
                   pl.BlockSpec(memory_space=pltpu.VMEM),
                   pl.BlockSpec(memory_space=pltpu.VMEM)),
    )(scr, dlt, anc, im_info)

    mesh = plsc.VectorSubcoreMesh(core_axis_name="c", subcore_axis_name="s")

    ilist = pl.kernel(
        _compact_kernel,
        mesh=mesh,
        compiler_params=pltpu.CompilerParams(needs_layout_passes=False),
        out_type=jax.ShapeDtypeStruct((B * _ILIST,), i32),
        scratch_types=[
            pltpu.VMEM((_CH,), i32),
            pltpu.VMEM((32,), i32),
            pltpu.VMEM((_CPAD,), i32),
            pltpu.VMEM((16,), i32),
            pltpu.VMEM((256,), i32),
            pltpu.VMEM((_CPAD,), i32),
            pltpu.VMEM((_ILIST,), i32),
            pltpu.VMEM_SHARED((2 * 16 * _CPAD,), i32),
            pltpu.VMEM_SHARED((2 * 16 * 16,), i32),
        ],
    )(keys.reshape(B * _N), thr.reshape(2 * B * _LANES))

    compact = pl.kernel(
        _gather_kernel,
        mesh=mesh,
        compiler_params=pltpu.CompilerParams(needs_layout_passes=False),
        out_type=jax.ShapeDtypeStruct((B * 5 * _CN,), f32),
        scratch_types=[
            pltpu.VMEM((_N,), f32),
            pltpu.VMEM((_CN,), i32),
            pltpu.VMEM((_CN,), f32),
        ],
    )(planes.reshape(5 * B * _N), ilist)

    pc = ilist.reshape(B, _ILIST)[:, :_CN].reshape(B, _CR, _LANES)

    rows = pl.pallas_call(
        _nms_kernel,
        out_shape=jax.ShapeDtypeStruct((_POST, _LANES), f32),
        in_specs=[pl.BlockSpec(memory_space=pltpu.VMEM)] * 2,
        out_specs=pl.BlockSpec(memory_space=pltpu.VMEM),
    )(compact.reshape(B, 5, _CR, _LANES), pc)

    boxes = jnp.transpose(rows[:, :4 * B].reshape(_POST, 4, B), (2, 0, 1))
    bcol = jnp.broadcast_to(jnp.arange(B, dtype=f32)[:, None, None], (B, _POST, 1))
    return jnp.concatenate([bcol, boxes], axis=2)
